# baseline (speedup 1.0000x reference)
import functools

import jax
import jax.numpy as jnp
from jax.experimental import pallas as pl


_TILE = 1024


def _fused_kernel(obs_ref, z_ref, w0_ref, b0_ref, w1_ref, b1_ref,
                  w2_ref, b2_ref, out_ref):
    h0 = jnp.maximum(
        jnp.dot(obs_ref[...], w0_ref[...], preferred_element_type=jnp.float32)
        + b0_ref[...], 0.0)
    h1 = jnp.maximum(
        jnp.dot(h0, w1_ref[...], preferred_element_type=jnp.float32)
        + b1_ref[...], 0.0)
    s = (jnp.dot(h1, w2_ref[...], preferred_element_type=jnp.float32)
         + b2_ref[...])
    m = z_ref[...]
    lane = jax.lax.broadcasted_iota(jnp.int32, (1, 16), 1)
    mask = (lane == m).astype(jnp.float32)
    x = jnp.sum(s[:, :16] * mask, axis=1, keepdims=True)
    y = jnp.sum(s[:, 16:32] * mask, axis=1, keepdims=True)
    out_ref[...] = jnp.concatenate([x, y], axis=1)


@functools.partial(jax.jit, static_argnames=())
def _run(obs_vec, z_logits, W0, b0, W1cat, b1cat, W2blk, b2cat):
    B = obs_vec.shape[0]
    tile = _TILE
    grid = (B // tile,)
    z2d = z_logits.reshape(B, 1)
    return pl.pallas_call(
        _fused_kernel,
        grid=grid,
        in_specs=[
            pl.BlockSpec((tile, 10), lambda i: (i, 0)),
            pl.BlockSpec((tile, 1), lambda i: (i, 0)),
            pl.BlockSpec((10, 32), lambda i: (0, 0)),
            pl.BlockSpec((1, 32), lambda i: (0, 0)),
            pl.BlockSpec((32, 512), lambda i: (0, 0)),
            pl.BlockSpec((1, 512), lambda i: (0, 0)),
            pl.BlockSpec((512, 32), lambda i: (0, 0)),
            pl.BlockSpec((1, 32), lambda i: (0, 0)),
        ],
        out_specs=pl.BlockSpec((tile, 2), lambda i: (i, 0)),
        out_shape=jax.ShapeDtypeStruct((B, 2), jnp.float32),
    )(obs_vec, z2d, W0, b0.reshape(1, 32), W1cat, b1cat, W2blk, b2cat)


def kernel(obs_vec, z_logits, best, W0, b0, Wx1, bx1, Wx2, bx2, Wy1, by1, Wy2, by2):
    n_modes = Wx1.shape[0]
    hid = Wx1.shape[2]
    W1x = Wx1.transpose(1, 0, 2).reshape(32, n_modes * hid)
    W1y = Wy1.transpose(1, 0, 2).reshape(32, n_modes * hid)
    W1cat = jnp.concatenate([W1x, W1y], axis=1)
    b1cat = jnp.concatenate([bx1.reshape(1, -1), by1.reshape(1, -1)], axis=1)
    ex = jnp.eye(n_modes, dtype=jnp.float32)
    w2x = Wx2[:, :, 0]
    w2y = Wy2[:, :, 0]
    blk_x = (ex[:, None, :] * w2x[:, :, None]).reshape(n_modes * hid, n_modes)
    blk_y = (ex[:, None, :] * w2y[:, :, None]).reshape(n_modes * hid, n_modes)
    zeros = jnp.zeros_like(blk_x)
    W2blk = jnp.concatenate(
        [jnp.concatenate([blk_x, zeros], axis=1),
         jnp.concatenate([zeros, blk_y], axis=1)], axis=0)
    b2cat = jnp.concatenate([bx2[:, 0], by2[:, 0]]).reshape(1, 32)

    actions = _run(obs_vec, z_logits, W0, b0, W1cat, b1cat, W2blk, b2cat)
    return (actions, z_logits)


# speedup vs baseline: 1.0181x; 1.0181x over previous
import functools

import jax
import jax.numpy as jnp
from jax.experimental import pallas as pl


_TILE = 1024


def _fused_kernel(obs_ref, z_ref, w0_ref, b0_ref, w1_ref, b1_ref,
                  w2_ref, b2_ref, out_ref):
    h0 = jnp.maximum(
        jnp.dot(obs_ref[...], w0_ref[...], preferred_element_type=jnp.float32)
        + b0_ref[...], 0.0).astype(jnp.bfloat16)
    h1 = jnp.maximum(
        jnp.dot(h0, w1_ref[...], preferred_element_type=jnp.float32)
        + b1_ref[...], 0.0).astype(jnp.bfloat16)
    s = (jnp.dot(h1, w2_ref[...], preferred_element_type=jnp.float32)
         + b2_ref[...])
    m = z_ref[...]
    lane = jax.lax.broadcasted_iota(jnp.int32, (1, 16), 1)
    mask = (lane == m).astype(jnp.float32)
    x = jnp.sum(s[:, :16] * mask, axis=1, keepdims=True)
    y = jnp.sum(s[:, 16:32] * mask, axis=1, keepdims=True)
    out_ref[...] = jnp.concatenate([x, y], axis=1)


@functools.partial(jax.jit, static_argnames=())
def _run(obs_vec, z_logits, W0, b0, W1cat, b1cat, W2blk, b2cat):
    B = obs_vec.shape[0]
    tile = _TILE
    grid = (B // tile,)
    z2d = z_logits.reshape(B, 1)
    return pl.pallas_call(
        _fused_kernel,
        grid=grid,
        in_specs=[
            pl.BlockSpec((tile, 10), lambda i: (i, 0)),
            pl.BlockSpec((tile, 1), lambda i: (i, 0)),
            pl.BlockSpec((10, 32), lambda i: (0, 0)),
            pl.BlockSpec((1, 32), lambda i: (0, 0)),
            pl.BlockSpec((32, 512), lambda i: (0, 0)),
            pl.BlockSpec((1, 512), lambda i: (0, 0)),
            pl.BlockSpec((512, 32), lambda i: (0, 0)),
            pl.BlockSpec((1, 32), lambda i: (0, 0)),
        ],
        out_specs=pl.BlockSpec((tile, 2), lambda i: (i, 0)),
        out_shape=jax.ShapeDtypeStruct((B, 2), jnp.float32),
    )(obs_vec.astype(jnp.bfloat16), z2d, W0.astype(jnp.bfloat16),
      b0.reshape(1, 32), W1cat.astype(jnp.bfloat16), b1cat,
      W2blk.astype(jnp.bfloat16), b2cat)


def kernel(obs_vec, z_logits, best, W0, b0, Wx1, bx1, Wx2, bx2, Wy1, by1, Wy2, by2):
    n_modes = Wx1.shape[0]
    hid = Wx1.shape[2]
    W1x = Wx1.transpose(1, 0, 2).reshape(32, n_modes * hid)
    W1y = Wy1.transpose(1, 0, 2).reshape(32, n_modes * hid)
    W1cat = jnp.concatenate([W1x, W1y], axis=1)
    b1cat = jnp.concatenate([bx1.reshape(1, -1), by1.reshape(1, -1)], axis=1)
    ex = jnp.eye(n_modes, dtype=jnp.float32)
    w2x = Wx2[:, :, 0]
    w2y = Wy2[:, :, 0]
    blk_x = (ex[:, None, :] * w2x[:, :, None]).reshape(n_modes * hid, n_modes)
    blk_y = (ex[:, None, :] * w2y[:, :, None]).reshape(n_modes * hid, n_modes)
    zeros = jnp.zeros_like(blk_x)
    W2blk = jnp.concatenate(
        [jnp.concatenate([blk_x, zeros], axis=1),
         jnp.concatenate([zeros, blk_y], axis=1)], axis=0)
    b2cat = jnp.concatenate([bx2[:, 0], by2[:, 0]]).reshape(1, 32)

    actions = _run(obs_vec, z_logits, W0, b0, W1cat, b1cat, W2blk, b2cat)
    return (actions, z_logits)


# speedup vs baseline: 1.5628x; 1.5350x over previous
import functools

import jax
import jax.numpy as jnp
from jax.experimental import pallas as pl


_TILE = 1024


def _fused_kernel(obs_ref, z_ref, w0_ref, b0_ref, w1_ref, b1_ref,
                  w2_ref, b2_ref, out_ref):
    h0 = jnp.maximum(
        jnp.dot(obs_ref[...], w0_ref[...], preferred_element_type=jnp.float32)
        + b0_ref[...], 0.0).astype(jnp.bfloat16)
    h1 = jnp.maximum(
        jnp.dot(h0, w1_ref[...], preferred_element_type=jnp.float32)
        + b1_ref[...], 0.0).astype(jnp.bfloat16)
    s = (jnp.dot(h1, w2_ref[...], preferred_element_type=jnp.float32)
         + b2_ref[...])
    m = z_ref[...]
    lane = jax.lax.broadcasted_iota(jnp.int32, (1, 32), 1)
    mask = ((lane & 15) == m).astype(jnp.float32)
    e = jnp.concatenate(
        [jnp.where(lane < 16, 1.0, 0.0).reshape(32, 1),
         jnp.where(lane >= 16, 1.0, 0.0).reshape(32, 1)], axis=1)
    out_ref[...] = jnp.dot(s * mask, e, preferred_element_type=jnp.float32)


@functools.partial(jax.jit, static_argnames=())
def _run(obs_vec, z_logits, W0, b0, W1cat, b1cat, W2blk, b2cat):
    B = obs_vec.shape[0]
    tile = _TILE
    grid = (B // tile,)
    z2d = z_logits.reshape(B, 1)
    return pl.pallas_call(
        _fused_kernel,
        grid=grid,
        in_specs=[
            pl.BlockSpec((tile, 10), lambda i: (i, 0)),
            pl.BlockSpec((tile, 1), lambda i: (i, 0)),
            pl.BlockSpec((10, 32), lambda i: (0, 0)),
            pl.BlockSpec((1, 32), lambda i: (0, 0)),
            pl.BlockSpec((32, 512), lambda i: (0, 0)),
            pl.BlockSpec((1, 512), lambda i: (0, 0)),
            pl.BlockSpec((512, 32), lambda i: (0, 0)),
            pl.BlockSpec((1, 32), lambda i: (0, 0)),
        ],
        out_specs=pl.BlockSpec((tile, 2), lambda i: (i, 0)),
        out_shape=jax.ShapeDtypeStruct((B, 2), jnp.float32),
    )(obs_vec.astype(jnp.bfloat16), z2d, W0.astype(jnp.bfloat16),
      b0.reshape(1, 32), W1cat.astype(jnp.bfloat16), b1cat,
      W2blk.astype(jnp.bfloat16), b2cat)


def kernel(obs_vec, z_logits, best, W0, b0, Wx1, bx1, Wx2, bx2, Wy1, by1, Wy2, by2):
    n_modes = Wx1.shape[0]
    hid = Wx1.shape[2]
    W1x = Wx1.transpose(1, 0, 2).reshape(32, n_modes * hid)
    W1y = Wy1.transpose(1, 0, 2).reshape(32, n_modes * hid)
    W1cat = jnp.concatenate([W1x, W1y], axis=1)
    b1cat = jnp.concatenate([bx1.reshape(1, -1), by1.reshape(1, -1)], axis=1)
    ex = jnp.eye(n_modes, dtype=jnp.float32)
    w2x = Wx2[:, :, 0]
    w2y = Wy2[:, :, 0]
    blk_x = (ex[:, None, :] * w2x[:, :, None]).reshape(n_modes * hid, n_modes)
    blk_y = (ex[:, None, :] * w2y[:, :, None]).reshape(n_modes * hid, n_modes)
    zeros = jnp.zeros_like(blk_x)
    W2blk = jnp.concatenate(
        [jnp.concatenate([blk_x, zeros], axis=1),
         jnp.concatenate([zeros, blk_y], axis=1)], axis=0)
    b2cat = jnp.concatenate([bx2[:, 0], by2[:, 0]]).reshape(1, 32)

    actions = _run(obs_vec, z_logits, W0, b0, W1cat, b1cat, W2blk, b2cat)
    return (actions, z_logits)


# speedup vs baseline: 1.6978x; 1.0864x over previous
import functools

import jax
import jax.numpy as jnp
from jax.experimental import pallas as pl


_TILE = 2048


def _fused_kernel(obs_ref, z_ref, w0_ref, b0_ref, w1_ref, b1_ref,
                  w2_ref, b2_ref, out_ref):
    h0 = jnp.maximum(
        jnp.dot(obs_ref[...], w0_ref[...], preferred_element_type=jnp.float32)
        + b0_ref[...], 0.0).astype(jnp.bfloat16)
    h1 = jnp.maximum(
        jnp.dot(h0, w1_ref[...], preferred_element_type=jnp.float32)
        .astype(jnp.bfloat16) + b1_ref[...], jnp.bfloat16(0))
    s = (jnp.dot(h1, w2_ref[...], preferred_element_type=jnp.float32)
         + b2_ref[...])
    m = z_ref[...]
    lane = jax.lax.broadcasted_iota(jnp.int32, (1, 32), 1)
    mask = ((lane & 15) == m).astype(jnp.float32)
    e = jnp.concatenate(
        [jnp.where(lane < 16, 1.0, 0.0).reshape(32, 1),
         jnp.where(lane >= 16, 1.0, 0.0).reshape(32, 1)], axis=1)
    out_ref[...] = jnp.dot(s * mask, e, preferred_element_type=jnp.float32)


@functools.partial(jax.jit, static_argnames=())
def _run(obs_vec, z_logits, W0, b0, W1cat, b1cat, W2blk, b2cat):
    B = obs_vec.shape[0]
    tile = _TILE
    grid = (B // tile,)
    z2d = z_logits.reshape(B, 1)
    return pl.pallas_call(
        _fused_kernel,
        grid=grid,
        in_specs=[
            pl.BlockSpec((tile, 10), lambda i: (i, 0)),
            pl.BlockSpec((tile, 1), lambda i: (i, 0)),
            pl.BlockSpec((10, 32), lambda i: (0, 0)),
            pl.BlockSpec((1, 32), lambda i: (0, 0)),
            pl.BlockSpec((32, 512), lambda i: (0, 0)),
            pl.BlockSpec((1, 512), lambda i: (0, 0)),
            pl.BlockSpec((512, 32), lambda i: (0, 0)),
            pl.BlockSpec((1, 32), lambda i: (0, 0)),
        ],
        out_specs=pl.BlockSpec((tile, 2), lambda i: (i, 0)),
        out_shape=jax.ShapeDtypeStruct((B, 2), jnp.float32),
    )(obs_vec.astype(jnp.bfloat16), z2d, W0.astype(jnp.bfloat16),
      b0.reshape(1, 32).astype(jnp.bfloat16), W1cat.astype(jnp.bfloat16),
      b1cat.astype(jnp.bfloat16), W2blk.astype(jnp.bfloat16), b2cat)


def kernel(obs_vec, z_logits, best, W0, b0, Wx1, bx1, Wx2, bx2, Wy1, by1, Wy2, by2):
    n_modes = Wx1.shape[0]
    hid = Wx1.shape[2]
    W1x = Wx1.transpose(1, 0, 2).reshape(32, n_modes * hid)
    W1y = Wy1.transpose(1, 0, 2).reshape(32, n_modes * hid)
    W1cat = jnp.concatenate([W1x, W1y], axis=1)
    b1cat = jnp.concatenate([bx1.reshape(1, -1), by1.reshape(1, -1)], axis=1)
    ex = jnp.eye(n_modes, dtype=jnp.float32)
    w2x = Wx2[:, :, 0]
    w2y = Wy2[:, :, 0]
    blk_x = (ex[:, None, :] * w2x[:, :, None]).reshape(n_modes * hid, n_modes)
    blk_y = (ex[:, None, :] * w2y[:, :, None]).reshape(n_modes * hid, n_modes)
    zeros = jnp.zeros_like(blk_x)
    W2blk = jnp.concatenate(
        [jnp.concatenate([blk_x, zeros], axis=1),
         jnp.concatenate([zeros, blk_y], axis=1)], axis=0)
    b2cat = jnp.concatenate([bx2[:, 0], by2[:, 0]]).reshape(1, 32)

    actions = _run(obs_vec, z_logits, W0, b0, W1cat, b1cat, W2blk, b2cat)
    return (actions, z_logits)


# speedup vs baseline: 1.7383x; 1.0238x over previous
import functools

import jax
import jax.numpy as jnp
from jax.experimental import pallas as pl


_TILE = 4096


def _fused_kernel(obs_ref, z_ref, w0_ref, b0_ref, w1_ref, b1_ref,
                  w2_ref, b2_ref, out_ref):
    h0 = jnp.maximum(
        jnp.dot(obs_ref[...], w0_ref[...], preferred_element_type=jnp.float32)
        + b0_ref[...], 0.0).astype(jnp.bfloat16)
    h1 = jnp.maximum(
        jnp.dot(h0, w1_ref[...], preferred_element_type=jnp.float32)
        .astype(jnp.bfloat16) + b1_ref[...], jnp.bfloat16(0))
    s = (jnp.dot(h1, w2_ref[...], preferred_element_type=jnp.float32)
         + b2_ref[...])
    m = z_ref[...]
    lane = jax.lax.broadcasted_iota(jnp.int32, (1, 32), 1)
    mask = ((lane & 15) == m).astype(jnp.float32)
    e = jnp.concatenate(
        [jnp.where(lane < 16, 1.0, 0.0).reshape(32, 1),
         jnp.where(lane >= 16, 1.0, 0.0).reshape(32, 1)], axis=1)
    out_ref[...] = jnp.dot(s * mask, e, preferred_element_type=jnp.float32)


@functools.partial(jax.jit, static_argnames=())
def _run(obs_vec, z_logits, W0, b0, W1cat, b1cat, W2blk, b2cat):
    B = obs_vec.shape[0]
    tile = _TILE
    grid = (B // tile,)
    z2d = z_logits.reshape(B, 1)
    return pl.pallas_call(
        _fused_kernel,
        grid=grid,
        in_specs=[
            pl.BlockSpec((tile, 10), lambda i: (i, 0)),
            pl.BlockSpec((tile, 1), lambda i: (i, 0)),
            pl.BlockSpec((10, 32), lambda i: (0, 0)),
            pl.BlockSpec((1, 32), lambda i: (0, 0)),
            pl.BlockSpec((32, 512), lambda i: (0, 0)),
            pl.BlockSpec((1, 512), lambda i: (0, 0)),
            pl.BlockSpec((512, 32), lambda i: (0, 0)),
            pl.BlockSpec((1, 32), lambda i: (0, 0)),
        ],
        out_specs=pl.BlockSpec((tile, 2), lambda i: (i, 0)),
        out_shape=jax.ShapeDtypeStruct((B, 2), jnp.float32),
    )(obs_vec.astype(jnp.bfloat16), z2d, W0.astype(jnp.bfloat16),
      b0.reshape(1, 32).astype(jnp.bfloat16), W1cat.astype(jnp.bfloat16),
      b1cat.astype(jnp.bfloat16), W2blk.astype(jnp.bfloat16), b2cat)


def kernel(obs_vec, z_logits, best, W0, b0, Wx1, bx1, Wx2, bx2, Wy1, by1, Wy2, by2):
    n_modes = Wx1.shape[0]
    hid = Wx1.shape[2]
    W1x = Wx1.transpose(1, 0, 2).reshape(32, n_modes * hid)
    W1y = Wy1.transpose(1, 0, 2).reshape(32, n_modes * hid)
    W1cat = jnp.concatenate([W1x, W1y], axis=1)
    b1cat = jnp.concatenate([bx1.reshape(1, -1), by1.reshape(1, -1)], axis=1)
    ex = jnp.eye(n_modes, dtype=jnp.float32)
    w2x = Wx2[:, :, 0]
    w2y = Wy2[:, :, 0]
    blk_x = (ex[:, None, :] * w2x[:, :, None]).reshape(n_modes * hid, n_modes)
    blk_y = (ex[:, None, :] * w2y[:, :, None]).reshape(n_modes * hid, n_modes)
    zeros = jnp.zeros_like(blk_x)
    W2blk = jnp.concatenate(
        [jnp.concatenate([blk_x, zeros], axis=1),
         jnp.concatenate([zeros, blk_y], axis=1)], axis=0)
    b2cat = jnp.concatenate([bx2[:, 0], by2[:, 0]]).reshape(1, 32)

    actions = _run(obs_vec, z_logits, W0, b0, W1cat, b1cat, W2blk, b2cat)
    return (actions, z_logits)
